# Initial kernel scaffold; baseline (speedup 1.0000x reference)
import jax
import jax.numpy as jnp
from jax.experimental import pallas as pl


def kernel(x, pos, type_pe_table, ln_gamma, ln_beta):
    raise NotImplementedError("write your pallas kernel here")


# speedup vs baseline: 1.5658x; 1.5658x over previous
import jax
import jax.numpy as jnp
from jax.experimental import pallas as pl
from jax.experimental.pallas import tpu as pltpu

_BLOCK_ROWS = 512
_EPS = 1e-12


def _ln_body(pos_ref, tab_ref, g_ref, b_ref, x_ref, o_ref):
    p = pos_ref[0]
    row = tab_ref[pl.ds(p, 1), :]
    xb = x_ref[...] + row
    mean = jnp.mean(xb, axis=1, keepdims=True)
    xc = xb - mean
    var = jnp.mean(xc * xc, axis=1, keepdims=True)
    inv = jax.lax.rsqrt(var + _EPS)
    o_ref[...] = (xc * inv) * g_ref[...] + b_ref[...]


def kernel(x, pos, type_pe_table, ln_gamma, ln_beta):
    B, L, D = x.shape
    rows = B * L
    x2 = x.reshape(rows, D)
    pos_arr = jnp.asarray(pos, dtype=jnp.int32).reshape(1)
    g2 = ln_gamma.reshape(1, D)
    b2 = ln_beta.reshape(1, D)
    n_blocks = pl.cdiv(rows, _BLOCK_ROWS)

    out = pl.pallas_call(
        _ln_body,
        grid=(n_blocks,),
        in_specs=[
            pl.BlockSpec(memory_space=pltpu.SMEM),
            pl.BlockSpec(type_pe_table.shape, lambda i: (0, 0)),
            pl.BlockSpec((1, D), lambda i: (0, 0)),
            pl.BlockSpec((1, D), lambda i: (0, 0)),
            pl.BlockSpec((_BLOCK_ROWS, D), lambda i: (i, 0)),
        ],
        out_specs=pl.BlockSpec((_BLOCK_ROWS, D), lambda i: (i, 0)),
        out_shape=jax.ShapeDtypeStruct((rows, D), x.dtype),
        compiler_params=pltpu.CompilerParams(
            dimension_semantics=("parallel",),
        ),
    )(pos_arr, type_pe_table, g2, b2, x2)
    return out.reshape(B, L, D)


# speedup vs baseline: 1.8583x; 1.1869x over previous
import jax
import jax.numpy as jnp
from jax.experimental import pallas as pl
from jax.experimental.pallas import tpu as pltpu

_BLOCK_ROWS = 1024
_EPS = 1e-12


def _ln_body(pos_ref, tab_ref, g_ref, b_ref, x_ref, o_ref):
    p = pos_ref[0]
    row = tab_ref[pl.ds(p, 1), :]
    xb = x_ref[...] + row
    mean = jnp.mean(xb, axis=1, keepdims=True)
    xc = xb - mean
    var = jnp.mean(xc * xc, axis=1, keepdims=True)
    inv = jax.lax.rsqrt(var + _EPS)
    o_ref[...] = (xc * inv) * g_ref[...] + b_ref[...]


def kernel(x, pos, type_pe_table, ln_gamma, ln_beta):
    B, L, D = x.shape
    rows = B * L
    x2 = x.reshape(rows, D)
    pos_arr = jnp.asarray(pos, dtype=jnp.int32).reshape(1)
    g2 = ln_gamma.reshape(1, D)
    b2 = ln_beta.reshape(1, D)
    n_blocks = pl.cdiv(rows, _BLOCK_ROWS)

    out = pl.pallas_call(
        _ln_body,
        grid=(n_blocks,),
        in_specs=[
            pl.BlockSpec(memory_space=pltpu.SMEM),
            pl.BlockSpec(type_pe_table.shape, lambda i: (0, 0)),
            pl.BlockSpec((1, D), lambda i: (0, 0)),
            pl.BlockSpec((1, D), lambda i: (0, 0)),
            pl.BlockSpec((_BLOCK_ROWS, D), lambda i: (i, 0)),
        ],
        out_specs=pl.BlockSpec((_BLOCK_ROWS, D), lambda i: (i, 0)),
        out_shape=jax.ShapeDtypeStruct((rows, D), x.dtype),
        compiler_params=pltpu.CompilerParams(
            dimension_semantics=("parallel",),
        ),
    )(pos_arr, type_pe_table, g2, b2, x2)
    return out.reshape(B, L, D)


# speedup vs baseline: 1.9088x; 1.0272x over previous
import jax
import jax.numpy as jnp
from jax.experimental import pallas as pl
from jax.experimental.pallas import tpu as pltpu

_BLOCK_ROWS = 2048
_EPS = 1e-12


def _ln_body(pos_ref, tab_ref, g_ref, b_ref, x_ref, o_ref):
    p = pos_ref[0]
    row = tab_ref[pl.ds(p, 1), :]
    xb = x_ref[...] + row
    mean = jnp.mean(xb, axis=1, keepdims=True)
    xc = xb - mean
    var = jnp.mean(xc * xc, axis=1, keepdims=True)
    inv = jax.lax.rsqrt(var + _EPS)
    o_ref[...] = (xc * inv) * g_ref[...] + b_ref[...]


def kernel(x, pos, type_pe_table, ln_gamma, ln_beta):
    B, L, D = x.shape
    rows = B * L
    x2 = x.reshape(rows, D)
    pos_arr = jnp.asarray(pos, dtype=jnp.int32).reshape(1)
    g2 = ln_gamma.reshape(1, D)
    b2 = ln_beta.reshape(1, D)
    n_blocks = pl.cdiv(rows, _BLOCK_ROWS)

    out = pl.pallas_call(
        _ln_body,
        grid=(n_blocks,),
        in_specs=[
            pl.BlockSpec(memory_space=pltpu.SMEM),
            pl.BlockSpec(type_pe_table.shape, lambda i: (0, 0)),
            pl.BlockSpec((1, D), lambda i: (0, 0)),
            pl.BlockSpec((1, D), lambda i: (0, 0)),
            pl.BlockSpec((_BLOCK_ROWS, D), lambda i: (i, 0)),
        ],
        out_specs=pl.BlockSpec((_BLOCK_ROWS, D), lambda i: (i, 0)),
        out_shape=jax.ShapeDtypeStruct((rows, D), x.dtype),
        compiler_params=pltpu.CompilerParams(
            dimension_semantics=("parallel",),
        ),
    )(pos_arr, type_pe_table, g2, b2, x2)
    return out.reshape(B, L, D)
